# Initial kernel scaffold; baseline (speedup 1.0000x reference)
import jax
import jax.numpy as jnp
from jax.experimental import pallas as pl


def kernel(inputs, token_table, dense_W, dense_b, pos_table):
    raise NotImplementedError("write your pallas kernel here")


# speedup vs baseline: 76.9031x; 76.9031x over previous
import jax
import jax.numpy as jnp
from jax import lax
from jax.experimental import pallas as pl
from jax.experimental.pallas import tpu as pltpu
from jax.experimental.pallas import tpu_sc as plsc

SEQ = 66
INNER = SEQ - 2
EMB = 128
XROW = SEQ * INNER
OROW = SEQ * EMB
NCORES = 2
NSUB = 16
NW = NCORES * NSUB
LANES = 16


def _sc_body(x_hbm, tok_hbm, w_hbm, b_hbm, pos_hbm, out_hbm,
             x_v, o_v, base_v, tok_v, w_v, b_v, sin0, sin1, sout0, sout1):
    batch = x_hbm.shape[0]
    bpt = batch // NW
    wid = lax.axis_index("s") * NCORES + lax.axis_index("c")
    first = wid * bpt

    pltpu.sync_copy(tok_hbm, tok_v)
    pltpu.sync_copy(w_hbm, w_v)
    pltpu.sync_copy(b_hbm, b_v)
    pltpu.sync_copy(pos_hbm, base_v)
    in_h = pltpu.make_async_copy(x_hbm.at[first], x_v.at[pl.ds(0, XROW)], sin0)
    in_h.start()

    lane = lax.iota(jnp.int32, LANES)
    even = (lane % 2) == 0

    def vdot(r, o):
        acc = jnp.zeros((LANES,), jnp.float32)
        for k in range(EMB // LANES):
            tv = tok_v[r, pl.ds(k * LANES, LANES)]
            wv = plsc.load_gather(w_v, [2 * (lane + k * LANES) + o])
            acc = acc + tv * wv
        return jnp.broadcast_to(jnp.sum(acc), (LANES,))

    v00 = vdot(0, 0)
    v01 = vdot(0, 1)
    v10 = vdot(1, 0)
    v11 = vdot(1, 1)
    bias_pat = plsc.load_gather(b_v, [lane % 2])
    v0_pat = jnp.where(even, v00, v01) + bias_pat
    scale_pat = jnp.where(even, v10 - v00, v11 - v01)

    def fold(t, carry):
        sl = pl.ds(t * LANES, LANES)
        base_v[sl] = base_v[sl] + v0_pat
        return carry
    lax.fori_loop(0, OROW // LANES, fold, 0)

    dup = lane // 2

    def compute(slot):
        off = slot * XROW

        def body(t, carry):
            xg = plsc.load_gather(x_v, [dup + (8 * t + off)])
            sl = pl.ds(t * LANES, LANES)
            o_v[slot, sl] = xg.astype(jnp.float32) * scale_pat + base_v[sl]
            return carry
        lax.fori_loop(0, OROW // LANES, body, 0)

    sin = (sin0, sin1)
    sout = (sout0, sout1)
    out_h = [None, None]
    for i in range(bpt):
        slot = i % 2
        in_h.wait()
        if i + 1 < bpt:
            in_h = pltpu.make_async_copy(
                x_hbm.at[first + i + 1],
                x_v.at[pl.ds((1 - slot) * XROW, XROW)], sin[1 - slot])
            in_h.start()
        if out_h[slot] is not None:
            out_h[slot].wait()
        compute(slot)
        out_h[slot] = pltpu.make_async_copy(
            o_v.at[slot], out_hbm.at[first + i], sout[slot])
        out_h[slot].start()
    out_h[0].wait()
    out_h[1].wait()


def kernel(inputs, token_table, dense_W, dense_b, pos_table):
    batch = inputs.shape[0]
    x = inputs.reshape(batch, XROW)
    pos = pos_table.reshape(OROW)
    b_pad = jnp.pad(dense_b.astype(jnp.float32), (0, LANES - dense_b.shape[0]))
    run = pl.kernel(
        _sc_body,
        out_type=jax.ShapeDtypeStruct((batch, OROW), jnp.float32),
        mesh=plsc.VectorSubcoreMesh(core_axis_name="c", subcore_axis_name="s"),
        compiler_params=pltpu.CompilerParams(needs_layout_passes=False),
        scratch_types=[
            pltpu.VMEM((2 * XROW,), jnp.int32),
            pltpu.VMEM((2, OROW), jnp.float32),
            pltpu.VMEM((OROW,), jnp.float32),
            pltpu.VMEM((2, EMB), jnp.float32),
            pltpu.VMEM((2 * EMB,), jnp.float32),
            pltpu.VMEM((LANES,), jnp.float32),
            pltpu.SemaphoreType.DMA,
            pltpu.SemaphoreType.DMA,
            pltpu.SemaphoreType.DMA,
            pltpu.SemaphoreType.DMA,
        ],
    )
    w_flat = dense_W.astype(jnp.float32).reshape(2 * EMB)
    out = run(x, token_table, w_flat, b_pad, pos)
    return out.reshape(batch, SEQ, EMB)


# speedup vs baseline: 140.8938x; 1.8321x over previous
import jax
import jax.numpy as jnp
from jax import lax
from jax.experimental import pallas as pl
from jax.experimental.pallas import tpu as pltpu
from jax.experimental.pallas import tpu_sc as plsc

SEQ = 66
INNER = SEQ - 2
EMB = 128
XROW = SEQ * INNER
OROW = SEQ * EMB
NCORES = 2
NSUB = 16
NW = NCORES * NSUB
LANES = 16


def _sc_body(x_hbm, tok_hbm, w_hbm, b_hbm, pos_hbm, out_hbm,
             x_v, o_v, base_v, tok_v, w_v, b_v, sin0, sin1, sout0, sout1):
    batch = x_hbm.shape[0]
    bpt = batch // NW
    wid = lax.axis_index("s") * NCORES + lax.axis_index("c")
    first = wid * bpt

    pltpu.sync_copy(tok_hbm, tok_v)
    pltpu.sync_copy(w_hbm, w_v)
    pltpu.sync_copy(b_hbm, b_v)
    pltpu.sync_copy(pos_hbm, base_v)
    in_h = pltpu.make_async_copy(x_hbm.at[first], x_v.at[pl.ds(0, XROW)], sin0)
    in_h.start()

    lane = lax.iota(jnp.int32, LANES)
    even = (lane % 2) == 0

    def vdot(r, o):
        acc = jnp.zeros((LANES,), jnp.float32)
        for k in range(EMB // LANES):
            tv = tok_v[r, pl.ds(k * LANES, LANES)]
            wv = plsc.load_gather(w_v, [2 * (lane + k * LANES) + o])
            acc = acc + tv * wv
        return jnp.broadcast_to(jnp.sum(acc), (LANES,))

    v00 = vdot(0, 0)
    v01 = vdot(0, 1)
    v10 = vdot(1, 0)
    v11 = vdot(1, 1)
    bias_pat = plsc.load_gather(b_v, [lane % 2])
    v0_pat = jnp.where(even, v00, v01) + bias_pat
    scale_pat = jnp.where(even, v10 - v00, v11 - v01)

    @plsc.parallel_loop(0, OROW // LANES, unroll=8)
    def _fold(t):
        sl = pl.ds(t * LANES, LANES)
        base_v[sl] = base_v[sl] + v0_pat

    dup = lane // 2

    def compute(slot):
        off = slot * XROW

        @plsc.parallel_loop(0, OROW // LANES, unroll=8)
        def _body(t):
            xg = plsc.load_gather(x_v, [dup + (8 * t + off)])
            sl = pl.ds(t * LANES, LANES)
            o_v[slot, sl] = xg.astype(jnp.float32) * scale_pat + base_v[sl]

    sin = (sin0, sin1)
    sout = (sout0, sout1)
    out_h = [None, None]
    for i in range(bpt):
        slot = i % 2
        in_h.wait()
        if i + 1 < bpt:
            in_h = pltpu.make_async_copy(
                x_hbm.at[first + i + 1],
                x_v.at[pl.ds((1 - slot) * XROW, XROW)], sin[1 - slot])
            in_h.start()
        if out_h[slot] is not None:
            out_h[slot].wait()
        compute(slot)
        out_h[slot] = pltpu.make_async_copy(
            o_v.at[slot], out_hbm.at[first + i], sout[slot])
        out_h[slot].start()
    out_h[0].wait()
    out_h[1].wait()


def kernel(inputs, token_table, dense_W, dense_b, pos_table):
    batch = inputs.shape[0]
    x = inputs.reshape(batch, XROW)
    pos = pos_table.reshape(OROW)
    b_pad = jnp.pad(dense_b.astype(jnp.float32), (0, LANES - dense_b.shape[0]))
    run = pl.kernel(
        _sc_body,
        out_type=jax.ShapeDtypeStruct((batch, OROW), jnp.float32),
        mesh=plsc.VectorSubcoreMesh(core_axis_name="c", subcore_axis_name="s"),
        compiler_params=pltpu.CompilerParams(needs_layout_passes=False),
        scratch_types=[
            pltpu.VMEM((2 * XROW,), jnp.int32),
            pltpu.VMEM((2, OROW), jnp.float32),
            pltpu.VMEM((OROW,), jnp.float32),
            pltpu.VMEM((2, EMB), jnp.float32),
            pltpu.VMEM((2 * EMB,), jnp.float32),
            pltpu.VMEM((LANES,), jnp.float32),
            pltpu.SemaphoreType.DMA,
            pltpu.SemaphoreType.DMA,
            pltpu.SemaphoreType.DMA,
            pltpu.SemaphoreType.DMA,
        ],
    )
    w_flat = dense_W.astype(jnp.float32).reshape(2 * EMB)
    out = run(x, token_table, w_flat, b_pad, pos)
    return out.reshape(batch, SEQ, EMB)
